# baseline (speedup 1.0000x reference)
import functools

import jax
import jax.numpy as jnp
from jax import lax
from jax.experimental import pallas as pl
from jax.experimental.pallas import tpu as pltpu
from jax.experimental.pallas import tpu_sc as plsc

N_NODES = 50000
N_USERS = 25000
N_ITEMS = 25000
FEAT = 128
HID = 64
HID2 = HID // 2
N_LAYERS = 3
N_EDGES = 800000
ALPHA = 1.0 / (N_LAYERS + 1)

NC = 2
NS = 16
HALF = N_NODES // NC
ROWS_PT = 1568
ACC = NS * ROWS_PT
K = 512
CHUNKS_PT = 102
G = 17
NSUP = 6
E_PAD = NS * CHUNKS_PT * K
EROWS = E_PAD // K
NB = 3
LA = 2
ZR = 112


def _sc_deg(dst2d):
    mesh = plsc.VectorSubcoreMesh(core_axis_name="c", subcore_axis_name="s",
                                  num_cores=NC, num_subcores=NS)

    @functools.partial(
        pl.kernel,
        out_type=(jax.ShapeDtypeStruct((NC * ACC,), jnp.float32),
                  jax.ShapeDtypeStruct((NC, EROWS, K), jnp.int32)),
        mesh=mesh,
        scratch_types=[
            pltpu.VMEM((G, K), jnp.int32),
            pltpu.VMEM((K,), jnp.float32),
            pltpu.VMEM((ROWS_PT,), jnp.float32),
            pltpu.VMEM_SHARED((ACC,), jnp.float32),
            pltpu.SemaphoreType.DMA((NB,)),
        ],
        compiler_params=pltpu.CompilerParams(use_tc_tiling_on_sc=False),
    )
    def k(dst_hbm, out_hbm, loc_hbm, locv, ones, zb, acc, ssem):
        c = lax.axis_index("c")
        s = lax.axis_index("s")
        lo = c * HALF

        @pl.loop(0, K // 16)
        def _(i):
            ones[pl.ds(i * 16, 16)] = jnp.full((16,), 1.0, jnp.float32)

        @pl.loop(0, ROWS_PT // 16)
        def _(i):
            zb[pl.ds(i * 16, 16)] = jnp.zeros((16,), jnp.float32)

        pltpu.sync_copy(zb, acc.at[pl.ds(s * ROWS_PT, ROWS_PT)])
        plsc.subcore_barrier()

        @pl.loop(0, NSUP)
        def _(g):
            base = s * CHUNKS_PT + g * G
            pltpu.sync_copy(dst_hbm.at[pl.ds(base, G)], locv)

            @pl.loop(0, G)
            def _(r):
                for q in range(K // 16):
                    d = locv[r, pl.ds(q * 16, 16)]
                    inh = (d >= lo) & (d < lo + HALF)
                    locv[r, pl.ds(q * 16, 16)] = jnp.where(inh, d - lo, HALF)

            pltpu.sync_copy(locv, loc_hbm.at[c, pl.ds(base, G)])
            sd = [None] * G
            for j in range(G):
                if j >= NB:
                    sd[j - NB].wait()
                sd[j] = pltpu.async_copy(ones, acc.at[locv.at[j]],
                                         ssem.at[j % NB], add=True)
            for j in range(G - NB, G):
                sd[j].wait()

        plsc.subcore_barrier()
        pltpu.sync_copy(acc.at[pl.ds(s * ROWS_PT, ROWS_PT)], zb)
        pltpu.sync_copy(zb, out_hbm.at[pl.ds(c * ACC + s * ROWS_PT, ROWS_PT)])

    return k(dst2d)


def _sc_prop(yh, src2d, loc2d):
    mesh = plsc.VectorSubcoreMesh(core_axis_name="c", subcore_axis_name="s",
                                  num_cores=NC, num_subcores=NS)

    @functools.partial(
        pl.kernel,
        out_type=jax.ShapeDtypeStruct((NC, ACC, HID2), jnp.float32),
        mesh=mesh,
        scratch_types=[
            pltpu.VMEM((G, K), jnp.int32),
            pltpu.VMEM((G, K), jnp.int32),
            pltpu.VMEM((NB, K, HID2), jnp.float32),
            pltpu.VMEM((ZR, HID2), jnp.float32),
            pltpu.VMEM_SHARED((ACC, HID2), jnp.float32),
            pltpu.SemaphoreType.DMA((NB,)),
            pltpu.SemaphoreType.DMA((NB,)),
        ],
        compiler_params=pltpu.CompilerParams(use_tc_tiling_on_sc=False),
    )
    def k(y_hbm, src_hbm, loc_hbm, out_hbm, srcv, locv, stage, zb, acc,
          gsem, ssem):
        c = lax.axis_index("c")
        s = lax.axis_index("s")

        @pl.loop(0, ZR)
        def _(r):
            for q in range(HID2 // 16):
                zb[r, pl.ds(q * 16, 16)] = jnp.zeros((16,), jnp.float32)

        for t in range(ROWS_PT // ZR):
            pltpu.sync_copy(zb, acc.at[pl.ds(s * ROWS_PT + t * ZR, ZR)])
        plsc.subcore_barrier()

        @pl.loop(0, NSUP)
        def _(g):
            base = s * CHUNKS_PT + g * G
            pltpu.sync_copy(src_hbm.at[pl.ds(base, G)], srcv)
            pltpu.sync_copy(loc_hbm.at[c, pl.ds(base, G)], locv)
            gd = [None] * G
            sd = [None] * G
            for j in range(-LA, G):
                ji = j + LA
                if 0 <= ji < G:
                    b = ji % NB
                    if ji >= NB:
                        sd[ji - NB].wait()
                    gd[ji] = pltpu.async_copy(y_hbm.at[srcv.at[ji]],
                                              stage.at[b], gsem.at[b])
                if j >= 0:
                    gd[j].wait()
                    sd[j] = pltpu.async_copy(stage.at[j % NB],
                                             acc.at[locv.at[j]],
                                             ssem.at[j % NB], add=True)
            for j in range(G - NB, G):
                sd[j].wait()

        plsc.subcore_barrier()
        for t in range(ROWS_PT // ZR):
            off = s * ROWS_PT + t * ZR
            pltpu.sync_copy(acc.at[pl.ds(off, ZR)], zb)
            pltpu.sync_copy(zb, out_hbm.at[c, pl.ds(off, ZR)])

    return k(yh, src2d, loc2d)


def _ln_block(x, g, b, eps=1e-5):
    m = jnp.mean(x, axis=-1, keepdims=True)
    v = jnp.mean((x - m) * (x - m), axis=-1, keepdims=True)
    return (x - m) / jnp.sqrt(v + eps) * g + b


def _tc_item(feat, emb_i, deg_i, W1, b1, g1, be1, W2, b2, g2, be2, W3, b3, mw):
    B = 1000
    grid = N_ITEMS // B

    def body(feat_ref, emb_ref, deg_ref, W1r, b1r, g1r, be1r, W2r, b2r, g2r,
             be2r, W3r, b3r, mwr, out0_ref, ylo_ref, yhi_ref):
        h = jnp.dot(feat_ref[...], W1r[...],
                    preferred_element_type=jnp.float32) + b1r[...]
        h = jnp.maximum(_ln_block(h, g1r[...], be1r[...]), 0.0)
        h = jnp.dot(h, W2r[...], preferred_element_type=jnp.float32) + b2r[...]
        h = jnp.maximum(_ln_block(h, g2r[...], be2r[...]), 0.0)
        h = jnp.dot(h, W3r[...], preferred_element_type=jnp.float32) + b3r[...]
        nrm = jnp.sqrt(jnp.sum(h * h, axis=-1, keepdims=True))
        meta = h / jnp.clip(nrm, 1e-12, None)
        e0 = emb_ref[...] + mwr[0, 0] * meta
        deg = deg_ref[...]
        dis = jnp.where(deg > 0, lax.rsqrt(deg), 0.0)
        out0_ref[...] = e0 * ALPHA
        y0 = e0 * dis
        ylo_ref[...] = y0[:, :HID2]
        yhi_ref[...] = y0[:, HID2:]

    full = lambda shp: pl.BlockSpec(shp, lambda i: (0, 0))
    return pl.pallas_call(
        body,
        grid=(grid,),
        in_specs=[
            pl.BlockSpec((B, FEAT), lambda i: (i, 0)),
            pl.BlockSpec((B, HID), lambda i: (i, 0)),
            pl.BlockSpec((B, 1), lambda i: (i, 0)),
            full((FEAT, 512)), full((1, 512)), full((1, 512)), full((1, 512)),
            full((512, HID)), full((1, HID)), full((1, HID)), full((1, HID)),
            full((HID, HID)), full((1, HID)), full((1, 1)),
        ],
        out_specs=[pl.BlockSpec((B, HID), lambda i: (i, 0)),
                   pl.BlockSpec((B, HID2), lambda i: (i, 0)),
                   pl.BlockSpec((B, HID2), lambda i: (i, 0))],
        out_shape=[jax.ShapeDtypeStruct((N_ITEMS, HID), jnp.float32),
                   jax.ShapeDtypeStruct((N_ITEMS, HID2), jnp.float32),
                   jax.ShapeDtypeStruct((N_ITEMS, HID2), jnp.float32)],
    )(feat, emb_i, deg_i, W1, b1.reshape(1, -1), g1.reshape(1, -1),
      be1.reshape(1, -1), W2, b2.reshape(1, -1), g2.reshape(1, -1),
      be2.reshape(1, -1), W3, b3.reshape(1, -1), mw.reshape(1, 1))


def _tc_user(emb_u, deg_u):
    B = 1000
    grid = N_USERS // B

    def body(emb_ref, deg_ref, out0_ref, ylo_ref, yhi_ref):
        e0 = emb_ref[...]
        deg = deg_ref[...]
        dis = jnp.where(deg > 0, lax.rsqrt(deg), 0.0)
        out0_ref[...] = e0 * ALPHA
        y0 = e0 * dis
        ylo_ref[...] = y0[:, :HID2]
        yhi_ref[...] = y0[:, HID2:]

    return pl.pallas_call(
        body,
        grid=(grid,),
        in_specs=[pl.BlockSpec((B, HID), lambda i: (i, 0)),
                  pl.BlockSpec((B, 1), lambda i: (i, 0))],
        out_specs=[pl.BlockSpec((B, HID), lambda i: (i, 0)),
                   pl.BlockSpec((B, HID2), lambda i: (i, 0)),
                   pl.BlockSpec((B, HID2), lambda i: (i, 0))],
        out_shape=[jax.ShapeDtypeStruct((N_USERS, HID), jnp.float32),
                   jax.ShapeDtypeStruct((N_USERS, HID2), jnp.float32),
                   jax.ShapeDtypeStruct((N_USERS, HID2), jnp.float32)],
    )(emb_u, deg_u)


def _tc_layer(a_lo, a_hi, deg, out_prev):
    B = 1000
    grid = N_NODES // B

    def body(alo_ref, ahi_ref, deg_ref, outp_ref, out_ref, ylo_ref, yhi_ref):
        deg = deg_ref[...]
        dis = jnp.where(deg > 0, lax.rsqrt(deg), 0.0)
        t_lo = alo_ref[...] * dis
        t_hi = ahi_ref[...] * dis
        t = jnp.concatenate([t_lo, t_hi], axis=1)
        out_ref[...] = outp_ref[...] + t * ALPHA
        ylo_ref[...] = t_lo * dis
        yhi_ref[...] = t_hi * dis

    return pl.pallas_call(
        body,
        grid=(grid,),
        in_specs=[pl.BlockSpec((B, HID2), lambda i: (i, 0)),
                  pl.BlockSpec((B, HID2), lambda i: (i, 0)),
                  pl.BlockSpec((B, 1), lambda i: (i, 0)),
                  pl.BlockSpec((B, HID), lambda i: (i, 0))],
        out_specs=[pl.BlockSpec((B, HID), lambda i: (i, 0)),
                   pl.BlockSpec((B, HID2), lambda i: (i, 0)),
                   pl.BlockSpec((B, HID2), lambda i: (i, 0))],
        out_shape=[jax.ShapeDtypeStruct((N_NODES, HID), jnp.float32),
                   jax.ShapeDtypeStruct((N_NODES, HID2), jnp.float32),
                   jax.ShapeDtypeStruct((N_NODES, HID2), jnp.float32)],
    )(a_lo, a_hi, deg, out_prev)


def kernel(edge_index, item_features, emb, W1, b1, g1, be1, W2, b2, g2, be2,
           W3, b3, meta_weight):
    src = edge_index[0].astype(jnp.int32)
    dst = edge_index[1].astype(jnp.int32)
    pad = E_PAD - N_EDGES
    src2d = jnp.concatenate([src, jnp.zeros((pad,), jnp.int32)]
                            ).reshape(EROWS, K)
    dst2d = jnp.concatenate([dst, jnp.full((pad,), -1, jnp.int32)]
                            ).reshape(EROWS, K)

    degp, loc2d = _sc_deg(dst2d)
    deg = jnp.concatenate([degp[:HALF], degp[ACC:ACC + HALF]]
                          ).reshape(N_NODES, 1)

    out0_i, ylo_i, yhi_i = _tc_item(item_features, emb[N_USERS:],
                                    deg[N_USERS:], W1, b1, g1, be1, W2, b2,
                                    g2, be2, W3, b3, meta_weight)
    out0_u, ylo_u, yhi_u = _tc_user(emb[:N_USERS], deg[:N_USERS])
    out = jnp.concatenate([out0_u, out0_i])
    y_lo = jnp.concatenate([ylo_u, ylo_i])
    y_hi = jnp.concatenate([yhi_u, yhi_i])

    for _ in range(N_LAYERS):
        ap_lo = _sc_prop(y_lo, src2d, loc2d)
        ap_hi = _sc_prop(y_hi, src2d, loc2d)
        a_lo = jnp.concatenate([ap_lo[0, :HALF], ap_lo[1, :HALF]])
        a_hi = jnp.concatenate([ap_hi[0, :HALF], ap_hi[1, :HALF]])
        out, y_lo, y_hi = _tc_layer(a_lo, a_hi, deg, out)
    return out


# speedup vs baseline: 1.8654x; 1.8654x over previous
import functools

import jax
import jax.numpy as jnp
from jax import lax
from jax.experimental import pallas as pl
from jax.experimental.pallas import tpu as pltpu
from jax.experimental.pallas import tpu_sc as plsc

N_NODES = 50000
N_USERS = 25000
N_ITEMS = 25000
FEAT = 128
HID = 64
HID2 = HID // 2
N_LAYERS = 3
N_EDGES = 800000
ALPHA = 1.0 / (N_LAYERS + 1)

NC = 2
NS = 16
E_PAD = 835584

HALF = N_NODES // NC
ROWS_PT = 1568
ACC = NS * ROWS_PT
KD = 512
CPT_D = 102
GD = 17
NSUP_D = 6
EROWS_D = E_PAD // KD

ROWS_PT2 = 3136
ACCF = NS * ROWS_PT2
K = 256
CPT = 204
G = 6
NSUP = 34
EROWS = E_PAD // K
NB = 3
LA = 2
ZR = 56


def _sc_deg(dst2d):
    mesh = plsc.VectorSubcoreMesh(core_axis_name="c", subcore_axis_name="s",
                                  num_cores=NC, num_subcores=NS)

    @functools.partial(
        pl.kernel,
        out_type=jax.ShapeDtypeStruct((NC * ACC,), jnp.float32),
        mesh=mesh,
        scratch_types=[
            pltpu.VMEM((GD, KD), jnp.int32),
            pltpu.VMEM((KD,), jnp.float32),
            pltpu.VMEM((ROWS_PT,), jnp.float32),
            pltpu.VMEM_SHARED((ACC,), jnp.float32),
            pltpu.SemaphoreType.DMA((NB,)),
        ],
        compiler_params=pltpu.CompilerParams(use_tc_tiling_on_sc=False),
    )
    def k(dst_hbm, out_hbm, locv, ones, zb, acc, ssem):
        c = lax.axis_index("c")
        s = lax.axis_index("s")
        lo = c * HALF

        @pl.loop(0, KD // 16)
        def _(i):
            ones[pl.ds(i * 16, 16)] = jnp.full((16,), 1.0, jnp.float32)

        @pl.loop(0, ROWS_PT // 16)
        def _(i):
            zb[pl.ds(i * 16, 16)] = jnp.zeros((16,), jnp.float32)

        pltpu.sync_copy(zb, acc.at[pl.ds(s * ROWS_PT, ROWS_PT)])
        plsc.subcore_barrier()

        @pl.loop(0, NSUP_D)
        def _(g):
            base = s * CPT_D + g * GD
            pltpu.sync_copy(dst_hbm.at[pl.ds(base, GD)], locv)

            @pl.loop(0, GD)
            def _(r):
                for q in range(KD // 16):
                    d = locv[r, pl.ds(q * 16, 16)]
                    inh = (d >= lo) & (d < lo + HALF)
                    locv[r, pl.ds(q * 16, 16)] = jnp.where(inh, d - lo, HALF)

            sd = [None] * GD
            for j in range(GD):
                if j >= NB:
                    sd[j - NB].wait()
                sd[j] = pltpu.async_copy(ones, acc.at[locv.at[j]],
                                         ssem.at[j % NB], add=True)
            for j in range(GD - NB, GD):
                sd[j].wait()

        plsc.subcore_barrier()
        pltpu.sync_copy(acc.at[pl.ds(s * ROWS_PT, ROWS_PT)], zb)
        pltpu.sync_copy(zb, out_hbm.at[pl.ds(c * ACC + s * ROWS_PT, ROWS_PT)])

    return k(dst2d)


def _sc_prop(y2, src2d, dst2d):
    mesh = plsc.VectorSubcoreMesh(core_axis_name="c", subcore_axis_name="s",
                                  num_cores=NC, num_subcores=NS)

    @functools.partial(
        pl.kernel,
        out_type=jax.ShapeDtypeStruct((NC, ACCF, HID2), jnp.float32),
        mesh=mesh,
        scratch_types=[
            pltpu.VMEM((G, K), jnp.int32),
            pltpu.VMEM((G, K), jnp.int32),
            pltpu.VMEM((NB, K, HID2), jnp.float32),
            pltpu.VMEM((ZR, HID2), jnp.float32),
            pltpu.VMEM_SHARED((ACCF, HID2), jnp.float32),
            pltpu.SemaphoreType.DMA((NB,)),
            pltpu.SemaphoreType.DMA((NB,)),
        ],
        compiler_params=pltpu.CompilerParams(use_tc_tiling_on_sc=False),
    )
    def k(y_hbm, src_hbm, dst_hbm, out_hbm, srcv, locv, stage, zb, acc,
          gsem, ssem):
        c = lax.axis_index("c")
        s = lax.axis_index("s")

        @pl.loop(0, ZR)
        def _(r):
            for q in range(HID2 // 16):
                zb[r, pl.ds(q * 16, 16)] = jnp.zeros((16,), jnp.float32)

        for t in range(ROWS_PT2 // ZR):
            pltpu.sync_copy(zb, acc.at[pl.ds(s * ROWS_PT2 + t * ZR, ZR)])
        plsc.subcore_barrier()

        @pl.loop(0, NSUP)
        def _(g):
            base = s * CPT + g * G
            pltpu.sync_copy(src_hbm.at[pl.ds(base, G)], srcv)
            pltpu.sync_copy(dst_hbm.at[pl.ds(base, G)], locv)

            @pl.loop(0, G)
            def _(r):
                for q in range(K // 16):
                    sv = srcv[r, pl.ds(q * 16, 16)]
                    srcv[r, pl.ds(q * 16, 16)] = sv * 2 + c
                    d = locv[r, pl.ds(q * 16, 16)]
                    locv[r, pl.ds(q * 16, 16)] = jnp.where(d >= 0, d, N_NODES)

            gd = [None] * G
            sd = [None] * G
            for j in range(-LA, G):
                ji = j + LA
                if 0 <= ji < G:
                    b = ji % NB
                    if ji >= NB:
                        sd[ji - NB].wait()
                    gd[ji] = pltpu.async_copy(y_hbm.at[srcv.at[ji]],
                                              stage.at[b], gsem.at[b])
                if j >= 0:
                    gd[j].wait()
                    sd[j] = pltpu.async_copy(stage.at[j % NB],
                                             acc.at[locv.at[j]],
                                             ssem.at[j % NB], add=True)
            for j in range(G - NB, G):
                sd[j].wait()

        plsc.subcore_barrier()
        for t in range(ROWS_PT2 // ZR):
            off = s * ROWS_PT2 + t * ZR
            pltpu.sync_copy(acc.at[pl.ds(off, ZR)], zb)
            pltpu.sync_copy(zb, out_hbm.at[c, pl.ds(off, ZR)])

    return k(y2, src2d, dst2d)


def _ln_block(x, g, b, eps=1e-5):
    m = jnp.mean(x, axis=-1, keepdims=True)
    v = jnp.mean((x - m) * (x - m), axis=-1, keepdims=True)
    return (x - m) / jnp.sqrt(v + eps) * g + b


def _tc_item(feat, emb_i, deg_i, W1, b1, g1, be1, W2, b2, g2, be2, W3, b3, mw):
    B = 1000
    grid = N_ITEMS // B

    def body(feat_ref, emb_ref, deg_ref, W1r, b1r, g1r, be1r, W2r, b2r, g2r,
             be2r, W3r, b3r, mwr, out0_ref, y0_ref):
        h = jnp.dot(feat_ref[...], W1r[...],
                    preferred_element_type=jnp.float32) + b1r[...]
        h = jnp.maximum(_ln_block(h, g1r[...], be1r[...]), 0.0)
        h = jnp.dot(h, W2r[...], preferred_element_type=jnp.float32) + b2r[...]
        h = jnp.maximum(_ln_block(h, g2r[...], be2r[...]), 0.0)
        h = jnp.dot(h, W3r[...], preferred_element_type=jnp.float32) + b3r[...]
        nrm = jnp.sqrt(jnp.sum(h * h, axis=-1, keepdims=True))
        meta = h / jnp.clip(nrm, 1e-12, None)
        e0 = emb_ref[...] + mwr[0, 0] * meta
        deg = deg_ref[...]
        dis = jnp.where(deg > 0, lax.rsqrt(deg), 0.0)
        out0_ref[...] = e0 * ALPHA
        y0_ref[...] = e0 * dis

    full = lambda shp: pl.BlockSpec(shp, lambda i: (0, 0))
    return pl.pallas_call(
        body,
        grid=(grid,),
        in_specs=[
            pl.BlockSpec((B, FEAT), lambda i: (i, 0)),
            pl.BlockSpec((B, HID), lambda i: (i, 0)),
            pl.BlockSpec((B, 1), lambda i: (i, 0)),
            full((FEAT, 512)), full((1, 512)), full((1, 512)), full((1, 512)),
            full((512, HID)), full((1, HID)), full((1, HID)), full((1, HID)),
            full((HID, HID)), full((1, HID)), full((1, 1)),
        ],
        out_specs=[pl.BlockSpec((B, HID), lambda i: (i, 0)),
                   pl.BlockSpec((B, HID), lambda i: (i, 0))],
        out_shape=[jax.ShapeDtypeStruct((N_ITEMS, HID), jnp.float32),
                   jax.ShapeDtypeStruct((N_ITEMS, HID), jnp.float32)],
    )(feat, emb_i, deg_i, W1, b1.reshape(1, -1), g1.reshape(1, -1),
      be1.reshape(1, -1), W2, b2.reshape(1, -1), g2.reshape(1, -1),
      be2.reshape(1, -1), W3, b3.reshape(1, -1), mw.reshape(1, 1))


def _tc_user(emb_u, deg_u):
    B = 1000
    grid = N_USERS // B

    def body(emb_ref, deg_ref, out0_ref, y0_ref):
        e0 = emb_ref[...]
        deg = deg_ref[...]
        dis = jnp.where(deg > 0, lax.rsqrt(deg), 0.0)
        out0_ref[...] = e0 * ALPHA
        y0_ref[...] = e0 * dis

    return pl.pallas_call(
        body,
        grid=(grid,),
        in_specs=[pl.BlockSpec((B, HID), lambda i: (i, 0)),
                  pl.BlockSpec((B, 1), lambda i: (i, 0))],
        out_specs=[pl.BlockSpec((B, HID), lambda i: (i, 0)),
                   pl.BlockSpec((B, HID), lambda i: (i, 0))],
        out_shape=[jax.ShapeDtypeStruct((N_USERS, HID), jnp.float32),
                   jax.ShapeDtypeStruct((N_USERS, HID), jnp.float32)],
    )(emb_u, deg_u)


def _tc_layer(a_lo, a_hi, deg, out_prev):
    B = 1000
    grid = N_NODES // B

    def body(alo_ref, ahi_ref, deg_ref, outp_ref, out_ref, y_ref):
        deg = deg_ref[...]
        dis = jnp.where(deg > 0, lax.rsqrt(deg), 0.0)
        t = jnp.concatenate([alo_ref[...], ahi_ref[...]], axis=1) * dis
        out_ref[...] = outp_ref[...] + t * ALPHA
        y_ref[...] = t * dis

    return pl.pallas_call(
        body,
        grid=(grid,),
        in_specs=[pl.BlockSpec((B, HID2), lambda i: (i, 0)),
                  pl.BlockSpec((B, HID2), lambda i: (i, 0)),
                  pl.BlockSpec((B, 1), lambda i: (i, 0)),
                  pl.BlockSpec((B, HID), lambda i: (i, 0))],
        out_specs=[pl.BlockSpec((B, HID), lambda i: (i, 0)),
                   pl.BlockSpec((B, HID), lambda i: (i, 0))],
        out_shape=[jax.ShapeDtypeStruct((N_NODES, HID), jnp.float32),
                   jax.ShapeDtypeStruct((N_NODES, HID), jnp.float32)],
    )(a_lo, a_hi, deg, out_prev)


def kernel(edge_index, item_features, emb, W1, b1, g1, be1, W2, b2, g2, be2,
           W3, b3, meta_weight):
    src = edge_index[0].astype(jnp.int32)
    dst = edge_index[1].astype(jnp.int32)
    pad = E_PAD - N_EDGES
    src_p = jnp.concatenate([src, jnp.zeros((pad,), jnp.int32)])
    dst_p = jnp.concatenate([dst, jnp.full((pad,), -1, jnp.int32)])

    degp = _sc_deg(dst_p.reshape(EROWS_D, KD))
    deg = jnp.concatenate([degp[:HALF], degp[ACC:ACC + HALF]]
                          ).reshape(N_NODES, 1)

    out0_i, y0_i = _tc_item(item_features, emb[N_USERS:], deg[N_USERS:],
                            W1, b1, g1, be1, W2, b2, g2, be2, W3, b3,
                            meta_weight)
    out0_u, y0_u = _tc_user(emb[:N_USERS], deg[:N_USERS])
    out = jnp.concatenate([out0_u, out0_i])
    y = jnp.concatenate([y0_u, y0_i])

    src2d = src_p.reshape(EROWS, K)
    dst2d = dst_p.reshape(EROWS, K)
    for _ in range(N_LAYERS):
        ap = _sc_prop(y.reshape(2 * N_NODES, HID2), src2d, dst2d)
        out, y = _tc_layer(ap[0, :N_NODES], ap[1, :N_NODES], deg, out)
    return out


# speedup vs baseline: 2.4904x; 1.3351x over previous
import functools

import jax
import jax.numpy as jnp
from jax import lax
from jax.experimental import pallas as pl
from jax.experimental.pallas import tpu as pltpu
from jax.experimental.pallas import tpu_sc as plsc

N_NODES = 50000
N_USERS = 25000
N_ITEMS = 25000
FEAT = 128
HID = 64
HID2 = HID // 2
N_LAYERS = 3
N_EDGES = 800000
ALPHA = 1.0 / (N_LAYERS + 1)

NC = 2
NS = 16
E_PAD = 835584

ROWS_PT = 3136
ACCF = NS * ROWS_PT

KD = 512
EROWS_D = E_PAD // KD
RPT_D = EROWS_D // (NC * NS)
GD = 17
NSUP_D = 3

K = 256
CPT = 204
G = 6
NSUP = 34
EROWS = E_PAD // K
NB = 3
LA = 2
ZR = 112
NUP = ROWS_PT // ZR


def _sc_deg(dst2d):
    mesh = plsc.VectorSubcoreMesh(core_axis_name="c", subcore_axis_name="s",
                                  num_cores=NC, num_subcores=NS)

    @functools.partial(
        pl.kernel,
        out_type=(jax.ShapeDtypeStruct((NC * ACCF,), jnp.float32),
                  jax.ShapeDtypeStruct((EROWS_D, KD), jnp.int32)),
        mesh=mesh,
        scratch_types=[
            pltpu.VMEM((GD, KD), jnp.int32),
            pltpu.VMEM((KD,), jnp.float32),
            pltpu.VMEM((ROWS_PT,), jnp.float32),
            pltpu.VMEM_SHARED((ACCF,), jnp.float32),
            pltpu.SemaphoreType.DMA((NB,)),
        ],
        compiler_params=pltpu.CompilerParams(use_tc_tiling_on_sc=False),
    )
    def k(dst_hbm, out_hbm, loc_hbm, locv, ones, zb, acc, ssem):
        c = lax.axis_index("c")
        s = lax.axis_index("s")
        w = c * NS + s

        @pl.loop(0, KD // 16)
        def _(i):
            ones[pl.ds(i * 16, 16)] = jnp.full((16,), 1.0, jnp.float32)

        @pl.loop(0, ROWS_PT // 16)
        def _(i):
            zb[pl.ds(i * 16, 16)] = jnp.zeros((16,), jnp.float32)

        pltpu.sync_copy(zb, acc.at[pl.ds(s * ROWS_PT, ROWS_PT)])
        plsc.subcore_barrier()

        @pl.loop(0, NSUP_D)
        def _(g):
            base = w * RPT_D + g * GD
            pltpu.sync_copy(dst_hbm.at[pl.ds(base, GD)], locv)

            @pl.loop(0, GD)
            def _(r):
                for q in range(KD // 16):
                    d = locv[r, pl.ds(q * 16, 16)]
                    locv[r, pl.ds(q * 16, 16)] = jnp.where(
                        d >= 0, d, N_NODES)

            pltpu.sync_copy(locv, loc_hbm.at[pl.ds(base, GD)])
            sd = [None] * GD
            for j in range(GD):
                if j >= NB:
                    sd[j - NB].wait()
                sd[j] = pltpu.async_copy(ones, acc.at[locv.at[j]],
                                         ssem.at[j % NB], add=True)
            for j in range(GD - NB, GD):
                sd[j].wait()

        plsc.subcore_barrier()
        pltpu.sync_copy(acc.at[pl.ds(s * ROWS_PT, ROWS_PT)], zb)
        pltpu.sync_copy(zb, out_hbm.at[pl.ds(c * ACCF + s * ROWS_PT,
                                             ROWS_PT)])

    return k(dst2d)


def _sc_prop(y2, src2d, loc2d, d2):
    mesh = plsc.VectorSubcoreMesh(core_axis_name="c", subcore_axis_name="s",
                                  num_cores=NC, num_subcores=NS)

    @functools.partial(
        pl.kernel,
        out_type=jax.ShapeDtypeStruct((NC * ACCF, HID2), jnp.float32),
        mesh=mesh,
        scratch_types=[
            pltpu.VMEM((G, K), jnp.int32),
            pltpu.VMEM((G, K), jnp.int32),
            pltpu.VMEM((NB, K, HID2), jnp.float32),
            pltpu.VMEM_SHARED((ACCF, HID2), jnp.float32),
            pltpu.SemaphoreType.DMA((NB,)),
            pltpu.SemaphoreType.DMA((NB,)),
        ],
        compiler_params=pltpu.CompilerParams(use_tc_tiling_on_sc=False),
    )
    def k(y_hbm, src_hbm, loc_hbm, d2_hbm, out_hbm, srcv, locv, stage, acc,
          gsem, ssem):
        c = lax.axis_index("c")
        s = lax.axis_index("s")
        roff = c * ACCF

        @pl.loop(0, ZR)
        def _(r):
            for q in range(HID2 // 16):
                stage[2, r, pl.ds(q * 16, 16)] = jnp.zeros((16,),
                                                           jnp.float32)

        for t in range(NUP):
            pltpu.sync_copy(stage.at[2, pl.ds(0, ZR)],
                            acc.at[pl.ds(s * ROWS_PT + t * ZR, ZR)])
        plsc.subcore_barrier()

        @pl.loop(0, NSUP)
        def _(g):
            base = s * CPT + g * G
            pltpu.sync_copy(src_hbm.at[pl.ds(base, G)], srcv)
            pltpu.sync_copy(loc_hbm.at[pl.ds(base, G)], locv)

            @pl.loop(0, G)
            def _(r):
                for q in range(K // 16):
                    sv = srcv[r, pl.ds(q * 16, 16)]
                    srcv[r, pl.ds(q * 16, 16)] = sv + roff

            gd = [None] * G
            sd = [None] * G
            for j in range(-LA, G):
                ji = j + LA
                if 0 <= ji < G:
                    b = ji % NB
                    if ji >= NB:
                        sd[ji - NB].wait()
                    gd[ji] = pltpu.async_copy(y_hbm.at[srcv.at[ji]],
                                              stage.at[b], gsem.at[b])
                if j >= 0:
                    gd[j].wait()
                    sd[j] = pltpu.async_copy(stage.at[j % NB],
                                             acc.at[locv.at[j]],
                                             ssem.at[j % NB], add=True)
            for j in range(G - NB, G):
                sd[j].wait()

        plsc.subcore_barrier()

        la = [None] * NUP
        ld = [None] * NUP
        wr = [None] * NUP
        off0 = s * ROWS_PT
        la[0] = pltpu.async_copy(acc.at[pl.ds(off0, ZR)],
                                 stage.at[0, pl.ds(0, ZR)], gsem.at[0])
        ld[0] = pltpu.async_copy(d2_hbm.at[pl.ds(off0, ZR)],
                                 stage.at[1, pl.ds(0, ZR)], gsem.at[1])
        for t in range(NUP):
            h = (t % 2) * 128
            la[t].wait()
            ld[t].wait()
            if t + 1 < NUP:
                off2 = off0 + (t + 1) * ZR
                h2 = ((t + 1) % 2) * 128
                la[t + 1] = pltpu.async_copy(acc.at[pl.ds(off2, ZR)],
                                             stage.at[0, pl.ds(h2, ZR)],
                                             gsem.at[0])
                ld[t + 1] = pltpu.async_copy(d2_hbm.at[pl.ds(off2, ZR)],
                                             stage.at[1, pl.ds(h2, ZR)],
                                             gsem.at[1])
            if t >= 2:
                wr[t - 2].wait()

            @pl.loop(0, ZR)
            def _(r):
                for q in range(HID2 // 16):
                    av = stage[0, h + r, pl.ds(q * 16, 16)]
                    dv = stage[1, h + r, pl.ds(q * 16, 16)]
                    stage[2, h + r, pl.ds(q * 16, 16)] = av * dv

            wr[t] = pltpu.async_copy(
                stage.at[2, pl.ds(h, ZR)],
                out_hbm.at[pl.ds(roff + off0 + t * ZR, ZR)],
                ssem.at[t % 2])
        wr[NUP - 2].wait()
        wr[NUP - 1].wait()

    return k(y2, src2d, loc2d, d2)


def _ln_block(x, g, b, eps=1e-5):
    m = jnp.mean(x, axis=-1, keepdims=True)
    v = jnp.mean((x - m) * (x - m), axis=-1, keepdims=True)
    return (x - m) / jnp.sqrt(v + eps) * g + b


def _tc_item(feat, emb_i, degA, degB, W1, b1, g1, be1, W2, b2, g2, be2,
             W3, b3, mw):
    B = 1000
    grid = N_ITEMS // B

    def body(feat_ref, emb_ref, dA_ref, dB_ref, W1r, b1r, g1r, be1r, W2r,
             b2r, g2r, be2r, W3r, b3r, mwr, out0_ref, ylo_ref, yhi_ref,
             d2_ref, deg_ref):
        h = jnp.dot(feat_ref[...], W1r[...],
                    preferred_element_type=jnp.float32) + b1r[...]
        h = jnp.maximum(_ln_block(h, g1r[...], be1r[...]), 0.0)
        h = jnp.dot(h, W2r[...], preferred_element_type=jnp.float32) + b2r[...]
        h = jnp.maximum(_ln_block(h, g2r[...], be2r[...]), 0.0)
        h = jnp.dot(h, W3r[...], preferred_element_type=jnp.float32) + b3r[...]
        nrm = jnp.sqrt(jnp.sum(h * h, axis=-1, keepdims=True))
        meta = h / jnp.clip(nrm, 1e-12, None)
        e0 = emb_ref[...] + mwr[0, 0] * meta
        deg = dA_ref[...] + dB_ref[...]
        dis = jnp.where(deg > 0, lax.rsqrt(deg), 0.0)
        out0_ref[...] = e0 * ALPHA
        y0 = e0 * dis
        ylo_ref[...] = y0[:, :HID2]
        yhi_ref[...] = y0[:, HID2:]
        d2_ref[...] = jnp.broadcast_to(dis * dis, (B, HID2))
        deg_ref[...] = deg

    full = lambda shp: pl.BlockSpec(shp, lambda i: (0, 0))
    return pl.pallas_call(
        body,
        grid=(grid,),
        in_specs=[
            pl.BlockSpec((B, FEAT), lambda i: (i, 0)),
            pl.BlockSpec((B, HID), lambda i: (i, 0)),
            pl.BlockSpec((B, 1), lambda i: (i, 0)),
            pl.BlockSpec((B, 1), lambda i: (i, 0)),
            full((FEAT, 512)), full((1, 512)), full((1, 512)), full((1, 512)),
            full((512, HID)), full((1, HID)), full((1, HID)), full((1, HID)),
            full((HID, HID)), full((1, HID)), full((1, 1)),
        ],
        out_specs=[pl.BlockSpec((B, HID), lambda i: (i, 0)),
                   pl.BlockSpec((B, HID2), lambda i: (i, 0)),
                   pl.BlockSpec((B, HID2), lambda i: (i, 0)),
                   pl.BlockSpec((B, HID2), lambda i: (i, 0)),
                   pl.BlockSpec((B, 1), lambda i: (i, 0))],
        out_shape=[jax.ShapeDtypeStruct((N_ITEMS, HID), jnp.float32),
                   jax.ShapeDtypeStruct((N_ITEMS, HID2), jnp.float32),
                   jax.ShapeDtypeStruct((N_ITEMS, HID2), jnp.float32),
                   jax.ShapeDtypeStruct((N_ITEMS, HID2), jnp.float32),
                   jax.ShapeDtypeStruct((N_ITEMS, 1), jnp.float32)],
    )(feat, emb_i, degA, degB, W1, b1.reshape(1, -1), g1.reshape(1, -1),
      be1.reshape(1, -1), W2, b2.reshape(1, -1), g2.reshape(1, -1),
      be2.reshape(1, -1), W3, b3.reshape(1, -1), mw.reshape(1, 1))


def _tc_user(emb_u, degA, degB):
    B = 1000
    grid = N_USERS // B

    def body(emb_ref, dA_ref, dB_ref, out0_ref, ylo_ref, yhi_ref, d2_ref,
             deg_ref):
        e0 = emb_ref[...]
        deg = dA_ref[...] + dB_ref[...]
        dis = jnp.where(deg > 0, lax.rsqrt(deg), 0.0)
        out0_ref[...] = e0 * ALPHA
        y0 = e0 * dis
        ylo_ref[...] = y0[:, :HID2]
        yhi_ref[...] = y0[:, HID2:]
        d2_ref[...] = jnp.broadcast_to(dis * dis, (B, HID2))
        deg_ref[...] = deg

    return pl.pallas_call(
        body,
        grid=(grid,),
        in_specs=[pl.BlockSpec((B, HID), lambda i: (i, 0)),
                  pl.BlockSpec((B, 1), lambda i: (i, 0)),
                  pl.BlockSpec((B, 1), lambda i: (i, 0))],
        out_specs=[pl.BlockSpec((B, HID), lambda i: (i, 0)),
                   pl.BlockSpec((B, HID2), lambda i: (i, 0)),
                   pl.BlockSpec((B, HID2), lambda i: (i, 0)),
                   pl.BlockSpec((B, HID2), lambda i: (i, 0)),
                   pl.BlockSpec((B, 1), lambda i: (i, 0))],
        out_shape=[jax.ShapeDtypeStruct((N_USERS, HID), jnp.float32),
                   jax.ShapeDtypeStruct((N_USERS, HID2), jnp.float32),
                   jax.ShapeDtypeStruct((N_USERS, HID2), jnp.float32),
                   jax.ShapeDtypeStruct((N_USERS, HID2), jnp.float32),
                   jax.ShapeDtypeStruct((N_USERS, 1), jnp.float32)],
    )(emb_u, degA, degB)


def _tc_fin(out0, deg, ylos, yhis):
    B = 1000
    grid = N_NODES // B

    def body(out0_ref, deg_ref, l1, l2, l3, h1, h2, h3, out_ref):
        sq = jnp.sqrt(deg_ref[...])
        lo = (l1[...] + l2[...] + l3[...]) * sq
        hi = (h1[...] + h2[...] + h3[...]) * sq
        out_ref[...] = out0_ref[...] + ALPHA * jnp.concatenate([lo, hi],
                                                               axis=1)

    bs64 = pl.BlockSpec((B, HID), lambda i: (i, 0))
    bs32 = pl.BlockSpec((B, HID2), lambda i: (i, 0))
    bs1 = pl.BlockSpec((B, 1), lambda i: (i, 0))
    return pl.pallas_call(
        body,
        grid=(grid,),
        in_specs=[bs64, bs1, bs32, bs32, bs32, bs32, bs32, bs32],
        out_specs=bs64,
        out_shape=jax.ShapeDtypeStruct((N_NODES, HID), jnp.float32),
    )(out0, deg, *ylos, *yhis)


def kernel(edge_index, item_features, emb, W1, b1, g1, be1, W2, b2, g2, be2,
           W3, b3, meta_weight):
    src = edge_index[0].astype(jnp.int32)
    dst = edge_index[1].astype(jnp.int32)
    pad = E_PAD - N_EDGES
    src_p = jnp.concatenate([src, jnp.zeros((pad,), jnp.int32)])
    dst_p = jnp.concatenate([dst, jnp.full((pad,), -1, jnp.int32)])

    degp, loc2d = _sc_deg(dst_p.reshape(EROWS_D, KD))
    degA = degp[:N_NODES].reshape(N_NODES, 1)
    degB = degp[ACCF:ACCF + N_NODES].reshape(N_NODES, 1)

    out0_i, ylo_i, yhi_i, d2_i, deg_i = _tc_item(
        item_features, emb[N_USERS:], degA[N_USERS:], degB[N_USERS:],
        W1, b1, g1, be1, W2, b2, g2, be2, W3, b3, meta_weight)
    out0_u, ylo_u, yhi_u, d2_u, deg_u = _tc_user(
        emb[:N_USERS], degA[:N_USERS], degB[:N_USERS])

    out0 = jnp.concatenate([out0_u, out0_i])
    deg = jnp.concatenate([deg_u, deg_i])
    padrows = ((0, ACCF - N_NODES), (0, 0))
    y2 = jnp.concatenate([
        jnp.pad(jnp.concatenate([ylo_u, ylo_i]), padrows),
        jnp.pad(jnp.concatenate([yhi_u, yhi_i]), padrows)])
    d2 = jnp.pad(jnp.concatenate([d2_u, d2_i]), padrows)

    src2d = src_p.reshape(EROWS, K)
    loc2d_p = loc2d.reshape(EROWS, K)
    ylos, yhis = [], []
    for _ in range(N_LAYERS):
        y2 = _sc_prop(y2, src2d, loc2d_p, d2)
        ylos.append(y2[:N_NODES])
        yhis.append(y2[ACCF:ACCF + N_NODES])
    return _tc_fin(out0, deg, ylos, yhis)


# speedup vs baseline: 2.5180x; 1.0111x over previous
import functools

import jax
import jax.numpy as jnp
from jax import lax
from jax.experimental import pallas as pl
from jax.experimental.pallas import tpu as pltpu
from jax.experimental.pallas import tpu_sc as plsc

N_NODES = 50000
N_USERS = 25000
N_ITEMS = 25000
FEAT = 128
HID = 64
HID2 = HID // 2
N_LAYERS = 3
N_EDGES = 800000
ALPHA = 1.0 / (N_LAYERS + 1)

NC = 2
NS = 16
E_PAD = 835584

ROWS_PT = 3136
ACCF = NS * ROWS_PT

KD = 512
EROWS_D = E_PAD // KD
RPT_D = EROWS_D // (NC * NS)
GD = 17
NSUP_D = 3

K = 192
CPT = 272
G = 8
NSUP = 34
EROWS = E_PAD // K
NB = 3
LA = 2
ZR = 56
HOFF = 96
NUP = ROWS_PT // ZR


def _sc_deg(dst2d):
    mesh = plsc.VectorSubcoreMesh(core_axis_name="c", subcore_axis_name="s",
                                  num_cores=NC, num_subcores=NS)

    @functools.partial(
        pl.kernel,
        out_type=(jax.ShapeDtypeStruct((NC * ACCF,), jnp.float32),
                  jax.ShapeDtypeStruct((EROWS_D, KD), jnp.int32)),
        mesh=mesh,
        scratch_types=[
            pltpu.VMEM((GD, KD), jnp.int32),
            pltpu.VMEM((KD,), jnp.float32),
            pltpu.VMEM((ROWS_PT,), jnp.float32),
            pltpu.VMEM_SHARED((ACCF,), jnp.float32),
            pltpu.SemaphoreType.DMA((NB,)),
        ],
        compiler_params=pltpu.CompilerParams(use_tc_tiling_on_sc=False),
    )
    def k(dst_hbm, out_hbm, loc_hbm, locv, ones, zb, acc, ssem):
        c = lax.axis_index("c")
        s = lax.axis_index("s")
        w = c * NS + s

        @pl.loop(0, KD // 16)
        def _(i):
            ones[pl.ds(i * 16, 16)] = jnp.full((16,), 1.0, jnp.float32)

        @pl.loop(0, ROWS_PT // 16)
        def _(i):
            zb[pl.ds(i * 16, 16)] = jnp.zeros((16,), jnp.float32)

        pltpu.sync_copy(zb, acc.at[pl.ds(s * ROWS_PT, ROWS_PT)])
        plsc.subcore_barrier()

        @pl.loop(0, NSUP_D)
        def _(g):
            base = w * RPT_D + g * GD
            pltpu.sync_copy(dst_hbm.at[pl.ds(base, GD)], locv)

            @pl.loop(0, GD)
            def _(r):
                for q in range(KD // 16):
                    d = locv[r, pl.ds(q * 16, 16)]
                    locv[r, pl.ds(q * 16, 16)] = jnp.where(
                        d >= 0, d, N_NODES)

            pltpu.sync_copy(locv, loc_hbm.at[pl.ds(base, GD)])
            sd = [None] * GD
            for j in range(GD):
                if j >= NB:
                    sd[j - NB].wait()
                sd[j] = pltpu.async_copy(ones, acc.at[locv.at[j]],
                                         ssem.at[j % NB], add=True)
            for j in range(GD - NB, GD):
                sd[j].wait()

        plsc.subcore_barrier()
        pltpu.sync_copy(acc.at[pl.ds(s * ROWS_PT, ROWS_PT)], zb)
        pltpu.sync_copy(zb, out_hbm.at[pl.ds(c * ACCF + s * ROWS_PT,
                                             ROWS_PT)])

    return k(dst2d)


def _sc_prop(y2, src2d, loc2d, d2):
    mesh = plsc.VectorSubcoreMesh(core_axis_name="c", subcore_axis_name="s",
                                  num_cores=NC, num_subcores=NS)

    @functools.partial(
        pl.kernel,
        out_type=jax.ShapeDtypeStruct((NC * ACCF, HID2), jnp.float32),
        mesh=mesh,
        scratch_types=[
            pltpu.VMEM((2, G, K), jnp.int32),
            pltpu.VMEM((2, G, K), jnp.int32),
            pltpu.VMEM((NB, K, HID2), jnp.float32),
            pltpu.VMEM_SHARED((ACCF, HID2), jnp.float32),
            pltpu.SemaphoreType.DMA((NB,)),
            pltpu.SemaphoreType.DMA((NB,)),
            pltpu.SemaphoreType.DMA,
            pltpu.SemaphoreType.DMA,
        ],
        compiler_params=pltpu.CompilerParams(use_tc_tiling_on_sc=False),
    )
    def k(y_hbm, src_hbm, loc_hbm, d2_hbm, out_hbm, srcv, locv, stage, acc,
          gsem, ssem, isemA, isemB):
        c = lax.axis_index("c")
        s = lax.axis_index("s")
        roff = c * ACCF

        @pl.loop(0, ZR)
        def _(r):
            for q in range(HID2 // 16):
                stage[2, r, pl.ds(q * 16, 16)] = jnp.zeros((16,),
                                                           jnp.float32)

        for t in range(NUP):
            pltpu.sync_copy(stage.at[2, pl.ds(0, ZR)],
                            acc.at[pl.ds(s * ROWS_PT + t * ZR, ZR)])
        plsc.subcore_barrier()

        pltpu.async_copy(src_hbm.at[pl.ds(s * CPT, G)], srcv.at[0], isemA)
        pltpu.async_copy(loc_hbm.at[pl.ds(s * CPT, G)], locv.at[0], isemB)

        @pl.loop(0, NSUP)
        def _(g):
            ib = lax.rem(g, 2)
            pltpu.make_async_copy(src_hbm.at[pl.ds(0, G)], srcv.at[ib],
                                  isemA).wait()
            pltpu.make_async_copy(loc_hbm.at[pl.ds(0, G)], locv.at[ib],
                                  isemB).wait()

            @pl.when(g < NSUP - 1)
            def _():
                nbase = s * CPT + (g + 1) * G
                pltpu.async_copy(src_hbm.at[pl.ds(nbase, G)],
                                 srcv.at[1 - ib], isemA)
                pltpu.async_copy(loc_hbm.at[pl.ds(nbase, G)],
                                 locv.at[1 - ib], isemB)

            @pl.loop(0, G)
            def _(r):
                for q in range(K // 16):
                    sv = srcv[ib, r, pl.ds(q * 16, 16)]
                    srcv[ib, r, pl.ds(q * 16, 16)] = sv + roff

            gd = [None] * G
            sd = [None] * G
            for j in range(-LA, G):
                ji = j + LA
                if 0 <= ji < G:
                    b = ji % NB
                    if ji >= NB:
                        sd[ji - NB].wait()
                    gd[ji] = pltpu.async_copy(y_hbm.at[srcv.at[ib, ji]],
                                              stage.at[b], gsem.at[b])
                if j >= 0:
                    gd[j].wait()
                    sd[j] = pltpu.async_copy(stage.at[j % NB],
                                             acc.at[locv.at[ib, j]],
                                             ssem.at[j % NB], add=True)
            for j in range(G - NB, G):
                sd[j].wait()

        plsc.subcore_barrier()

        la = [None] * NUP
        ld = [None] * NUP
        wr = [None] * NUP
        off0 = s * ROWS_PT
        la[0] = pltpu.async_copy(acc.at[pl.ds(off0, ZR)],
                                 stage.at[0, pl.ds(0, ZR)], gsem.at[0])
        ld[0] = pltpu.async_copy(d2_hbm.at[pl.ds(off0, ZR)],
                                 stage.at[1, pl.ds(0, ZR)], gsem.at[1])
        for t in range(NUP):
            h = (t % 2) * HOFF
            la[t].wait()
            ld[t].wait()
            if t + 1 < NUP:
                off2 = off0 + (t + 1) * ZR
                h2 = ((t + 1) % 2) * HOFF
                la[t + 1] = pltpu.async_copy(acc.at[pl.ds(off2, ZR)],
                                             stage.at[0, pl.ds(h2, ZR)],
                                             gsem.at[0])
                ld[t + 1] = pltpu.async_copy(d2_hbm.at[pl.ds(off2, ZR)],
                                             stage.at[1, pl.ds(h2, ZR)],
                                             gsem.at[1])
            if t >= 2:
                wr[t - 2].wait()

            @pl.loop(0, ZR)
            def _(r):
                for q in range(HID2 // 16):
                    av = stage[0, h + r, pl.ds(q * 16, 16)]
                    dv = stage[1, h + r, pl.ds(q * 16, 16)]
                    stage[2, h + r, pl.ds(q * 16, 16)] = av * dv

            wr[t] = pltpu.async_copy(
                stage.at[2, pl.ds(h, ZR)],
                out_hbm.at[pl.ds(roff + off0 + t * ZR, ZR)],
                ssem.at[t % 2])
        wr[NUP - 2].wait()
        wr[NUP - 1].wait()

    return k(y2, src2d, loc2d, d2)


def _ln_block(x, g, b, eps=1e-5):
    m = jnp.mean(x, axis=-1, keepdims=True)
    v = jnp.mean((x - m) * (x - m), axis=-1, keepdims=True)
    return (x - m) / jnp.sqrt(v + eps) * g + b


def _tc_item(feat, emb_i, degA, degB, W1, b1, g1, be1, W2, b2, g2, be2,
             W3, b3, mw):
    B = 1000
    grid = N_ITEMS // B

    def body(feat_ref, emb_ref, dA_ref, dB_ref, W1r, b1r, g1r, be1r, W2r,
             b2r, g2r, be2r, W3r, b3r, mwr, out0_ref, ylo_ref, yhi_ref,
             d2_ref, deg_ref):
        h = jnp.dot(feat_ref[...], W1r[...],
                    preferred_element_type=jnp.float32) + b1r[...]
        h = jnp.maximum(_ln_block(h, g1r[...], be1r[...]), 0.0)
        h = jnp.dot(h, W2r[...], preferred_element_type=jnp.float32) + b2r[...]
        h = jnp.maximum(_ln_block(h, g2r[...], be2r[...]), 0.0)
        h = jnp.dot(h, W3r[...], preferred_element_type=jnp.float32) + b3r[...]
        nrm = jnp.sqrt(jnp.sum(h * h, axis=-1, keepdims=True))
        meta = h / jnp.clip(nrm, 1e-12, None)
        e0 = emb_ref[...] + mwr[0, 0] * meta
        deg = dA_ref[...] + dB_ref[...]
        dis = jnp.where(deg > 0, lax.rsqrt(deg), 0.0)
        out0_ref[...] = e0 * ALPHA
        y0 = e0 * dis
        ylo_ref[...] = y0[:, :HID2]
        yhi_ref[...] = y0[:, HID2:]
        d2_ref[...] = jnp.broadcast_to(dis * dis, (B, HID2))
        deg_ref[...] = deg

    full = lambda shp: pl.BlockSpec(shp, lambda i: (0, 0))
    return pl.pallas_call(
        body,
        grid=(grid,),
        in_specs=[
            pl.BlockSpec((B, FEAT), lambda i: (i, 0)),
            pl.BlockSpec((B, HID), lambda i: (i, 0)),
            pl.BlockSpec((B, 1), lambda i: (i, 0)),
            pl.BlockSpec((B, 1), lambda i: (i, 0)),
            full((FEAT, 512)), full((1, 512)), full((1, 512)), full((1, 512)),
            full((512, HID)), full((1, HID)), full((1, HID)), full((1, HID)),
            full((HID, HID)), full((1, HID)), full((1, 1)),
        ],
        out_specs=[pl.BlockSpec((B, HID), lambda i: (i, 0)),
                   pl.BlockSpec((B, HID2), lambda i: (i, 0)),
                   pl.BlockSpec((B, HID2), lambda i: (i, 0)),
                   pl.BlockSpec((B, HID2), lambda i: (i, 0)),
                   pl.BlockSpec((B, 1), lambda i: (i, 0))],
        out_shape=[jax.ShapeDtypeStruct((N_ITEMS, HID), jnp.float32),
                   jax.ShapeDtypeStruct((N_ITEMS, HID2), jnp.float32),
                   jax.ShapeDtypeStruct((N_ITEMS, HID2), jnp.float32),
                   jax.ShapeDtypeStruct((N_ITEMS, HID2), jnp.float32),
                   jax.ShapeDtypeStruct((N_ITEMS, 1), jnp.float32)],
    )(feat, emb_i, degA, degB, W1, b1.reshape(1, -1), g1.reshape(1, -1),
      be1.reshape(1, -1), W2, b2.reshape(1, -1), g2.reshape(1, -1),
      be2.reshape(1, -1), W3, b3.reshape(1, -1), mw.reshape(1, 1))


def _tc_user(emb_u, degA, degB):
    B = 1000
    grid = N_USERS // B

    def body(emb_ref, dA_ref, dB_ref, out0_ref, ylo_ref, yhi_ref, d2_ref,
             deg_ref):
        e0 = emb_ref[...]
        deg = dA_ref[...] + dB_ref[...]
        dis = jnp.where(deg > 0, lax.rsqrt(deg), 0.0)
        out0_ref[...] = e0 * ALPHA
        y0 = e0 * dis
        ylo_ref[...] = y0[:, :HID2]
        yhi_ref[...] = y0[:, HID2:]
        d2_ref[...] = jnp.broadcast_to(dis * dis, (B, HID2))
        deg_ref[...] = deg

    return pl.pallas_call(
        body,
        grid=(grid,),
        in_specs=[pl.BlockSpec((B, HID), lambda i: (i, 0)),
                  pl.BlockSpec((B, 1), lambda i: (i, 0)),
                  pl.BlockSpec((B, 1), lambda i: (i, 0))],
        out_specs=[pl.BlockSpec((B, HID), lambda i: (i, 0)),
                   pl.BlockSpec((B, HID2), lambda i: (i, 0)),
                   pl.BlockSpec((B, HID2), lambda i: (i, 0)),
                   pl.BlockSpec((B, HID2), lambda i: (i, 0)),
                   pl.BlockSpec((B, 1), lambda i: (i, 0))],
        out_shape=[jax.ShapeDtypeStruct((N_USERS, HID), jnp.float32),
                   jax.ShapeDtypeStruct((N_USERS, HID2), jnp.float32),
                   jax.ShapeDtypeStruct((N_USERS, HID2), jnp.float32),
                   jax.ShapeDtypeStruct((N_USERS, HID2), jnp.float32),
                   jax.ShapeDtypeStruct((N_USERS, 1), jnp.float32)],
    )(emb_u, degA, degB)


def _tc_fin(out0, deg, ylos, yhis):
    B = 1000
    grid = N_NODES // B

    def body(out0_ref, deg_ref, l1, l2, l3, h1, h2, h3, out_ref):
        sq = jnp.sqrt(deg_ref[...])
        lo = (l1[...] + l2[...] + l3[...]) * sq
        hi = (h1[...] + h2[...] + h3[...]) * sq
        out_ref[...] = out0_ref[...] + ALPHA * jnp.concatenate([lo, hi],
                                                               axis=1)

    bs64 = pl.BlockSpec((B, HID), lambda i: (i, 0))
    bs32 = pl.BlockSpec((B, HID2), lambda i: (i, 0))
    bs1 = pl.BlockSpec((B, 1), lambda i: (i, 0))
    return pl.pallas_call(
        body,
        grid=(grid,),
        in_specs=[bs64, bs1, bs32, bs32, bs32, bs32, bs32, bs32],
        out_specs=bs64,
        out_shape=jax.ShapeDtypeStruct((N_NODES, HID), jnp.float32),
    )(out0, deg, *ylos, *yhis)


def kernel(edge_index, item_features, emb, W1, b1, g1, be1, W2, b2, g2, be2,
           W3, b3, meta_weight):
    src = edge_index[0].astype(jnp.int32)
    dst = edge_index[1].astype(jnp.int32)
    pad = E_PAD - N_EDGES
    src_p = jnp.concatenate([src, jnp.zeros((pad,), jnp.int32)])
    dst_p = jnp.concatenate([dst, jnp.full((pad,), -1, jnp.int32)])

    degp, loc2d = _sc_deg(dst_p.reshape(EROWS_D, KD))
    degA = degp[:N_NODES].reshape(N_NODES, 1)
    degB = degp[ACCF:ACCF + N_NODES].reshape(N_NODES, 1)

    out0_i, ylo_i, yhi_i, d2_i, deg_i = _tc_item(
        item_features, emb[N_USERS:], degA[N_USERS:], degB[N_USERS:],
        W1, b1, g1, be1, W2, b2, g2, be2, W3, b3, meta_weight)
    out0_u, ylo_u, yhi_u, d2_u, deg_u = _tc_user(
        emb[:N_USERS], degA[:N_USERS], degB[:N_USERS])

    out0 = jnp.concatenate([out0_u, out0_i])
    deg = jnp.concatenate([deg_u, deg_i])
    padrows = ((0, ACCF - N_NODES), (0, 0))
    y2 = jnp.concatenate([
        jnp.pad(jnp.concatenate([ylo_u, ylo_i]), padrows),
        jnp.pad(jnp.concatenate([yhi_u, yhi_i]), padrows)])
    d2 = jnp.pad(jnp.concatenate([d2_u, d2_i]), padrows)

    src2d = src_p.reshape(EROWS, K)
    loc2d_p = loc2d.reshape(EROWS, K)
    ylos, yhis = [], []
    for _ in range(N_LAYERS):
        y2 = _sc_prop(y2, src2d, loc2d_p, d2)
        ylos.append(y2[:N_NODES])
        yhis.append(y2[ACCF:ACCF + N_NODES])
    return _tc_fin(out0, deg, ylos, yhis)


# speedup vs baseline: 2.5625x; 1.0177x over previous
import functools

import jax
import jax.numpy as jnp
from jax import lax
from jax.experimental import pallas as pl
from jax.experimental.pallas import tpu as pltpu
from jax.experimental.pallas import tpu_sc as plsc

N_NODES = 50000
N_USERS = 25000
N_ITEMS = 25000
FEAT = 128
HID = 64
HID2 = HID // 2
N_LAYERS = 3
N_EDGES = 800000
ALPHA = 1.0 / (N_LAYERS + 1)

NC = 2
NS = 16
E_PAD = 835584

ROWS_PT = 3136
ACCF = NS * ROWS_PT

KD = 512
EROWS_D = E_PAD // KD
RPT_D = EROWS_D // (NC * NS)
GD = 17
NSUP_D = 3

K = 128
CPT = 408
G = 24
NSUP = 17
EROWS = E_PAD // K
NB = 4
LA = 3
ZR = 56
HOFF = 64
NUP = ROWS_PT // ZR


def _sc_deg(dst2d):
    mesh = plsc.VectorSubcoreMesh(core_axis_name="c", subcore_axis_name="s",
                                  num_cores=NC, num_subcores=NS)

    @functools.partial(
        pl.kernel,
        out_type=(jax.ShapeDtypeStruct((NC * ACCF,), jnp.float32),
                  jax.ShapeDtypeStruct((EROWS_D, KD), jnp.int32)),
        mesh=mesh,
        scratch_types=[
            pltpu.VMEM((GD, KD), jnp.int32),
            pltpu.VMEM((KD,), jnp.float32),
            pltpu.VMEM((ROWS_PT,), jnp.float32),
            pltpu.VMEM_SHARED((ACCF,), jnp.float32),
            pltpu.SemaphoreType.DMA((NB,)),
        ],
        compiler_params=pltpu.CompilerParams(use_tc_tiling_on_sc=False),
    )
    def k(dst_hbm, out_hbm, loc_hbm, locv, ones, zb, acc, ssem):
        c = lax.axis_index("c")
        s = lax.axis_index("s")
        w = c * NS + s

        @pl.loop(0, KD // 16)
        def _(i):
            ones[pl.ds(i * 16, 16)] = jnp.full((16,), 1.0, jnp.float32)

        @pl.loop(0, ROWS_PT // 16)
        def _(i):
            zb[pl.ds(i * 16, 16)] = jnp.zeros((16,), jnp.float32)

        pltpu.sync_copy(zb, acc.at[pl.ds(s * ROWS_PT, ROWS_PT)])
        plsc.subcore_barrier()

        @pl.loop(0, NSUP_D)
        def _(g):
            base = w * RPT_D + g * GD
            pltpu.sync_copy(dst_hbm.at[pl.ds(base, GD)], locv)

            @pl.loop(0, GD)
            def _(r):
                for q in range(KD // 16):
                    d = locv[r, pl.ds(q * 16, 16)]
                    locv[r, pl.ds(q * 16, 16)] = jnp.where(
                        d >= 0, d, N_NODES)

            pltpu.sync_copy(locv, loc_hbm.at[pl.ds(base, GD)])
            sd = [None] * GD
            for j in range(GD):
                if j >= NB:
                    sd[j - NB].wait()
                sd[j] = pltpu.async_copy(ones, acc.at[locv.at[j]],
                                         ssem.at[j % NB], add=True)
            for j in range(GD - NB, GD):
                sd[j].wait()

        plsc.subcore_barrier()
        pltpu.sync_copy(acc.at[pl.ds(s * ROWS_PT, ROWS_PT)], zb)
        pltpu.sync_copy(zb, out_hbm.at[pl.ds(c * ACCF + s * ROWS_PT,
                                             ROWS_PT)])

    return k(dst2d)


def _sc_prop(y2, src2d, loc2d, d2):
    mesh = plsc.VectorSubcoreMesh(core_axis_name="c", subcore_axis_name="s",
                                  num_cores=NC, num_subcores=NS)

    @functools.partial(
        pl.kernel,
        out_type=jax.ShapeDtypeStruct((NC * ACCF, HID2), jnp.float32),
        mesh=mesh,
        scratch_types=[
            pltpu.VMEM((2, G, K), jnp.int32),
            pltpu.VMEM((2, G, K), jnp.int32),
            pltpu.VMEM((NB, K, HID2), jnp.float32),
            pltpu.VMEM_SHARED((ACCF, HID2), jnp.float32),
            pltpu.SemaphoreType.DMA((NB,)),
            pltpu.SemaphoreType.DMA((NB,)),
            pltpu.SemaphoreType.DMA,
            pltpu.SemaphoreType.DMA,
        ],
        compiler_params=pltpu.CompilerParams(use_tc_tiling_on_sc=False),
    )
    def k(y_hbm, src_hbm, loc_hbm, d2_hbm, out_hbm, srcv, locv, stage, acc,
          gsem, ssem, isemA, isemB):
        c = lax.axis_index("c")
        s = lax.axis_index("s")
        roff = c * ACCF

        @pl.loop(0, ZR)
        def _(r):
            for q in range(HID2 // 16):
                stage[2, r, pl.ds(q * 16, 16)] = jnp.zeros((16,),
                                                           jnp.float32)

        for t in range(NUP):
            pltpu.sync_copy(stage.at[2, pl.ds(0, ZR)],
                            acc.at[pl.ds(s * ROWS_PT + t * ZR, ZR)])
        plsc.subcore_barrier()

        pltpu.async_copy(src_hbm.at[pl.ds(s * CPT, G)], srcv.at[0], isemA)
        pltpu.async_copy(loc_hbm.at[pl.ds(s * CPT, G)], locv.at[0], isemB)

        @pl.loop(0, NSUP)
        def _(g):
            ib = lax.rem(g, 2)
            pltpu.make_async_copy(src_hbm.at[pl.ds(0, G)], srcv.at[ib],
                                  isemA).wait()
            pltpu.make_async_copy(loc_hbm.at[pl.ds(0, G)], locv.at[ib],
                                  isemB).wait()

            @pl.when(g < NSUP - 1)
            def _():
                nbase = s * CPT + (g + 1) * G
                pltpu.async_copy(src_hbm.at[pl.ds(nbase, G)],
                                 srcv.at[1 - ib], isemA)
                pltpu.async_copy(loc_hbm.at[pl.ds(nbase, G)],
                                 locv.at[1 - ib], isemB)

            @pl.loop(0, G)
            def _(r):
                for q in range(K // 16):
                    sv = srcv[ib, r, pl.ds(q * 16, 16)]
                    srcv[ib, r, pl.ds(q * 16, 16)] = sv + roff

            gd = [None] * G
            sd = [None] * G
            for j in range(-LA, G):
                ji = j + LA
                if 0 <= ji < G:
                    b = ji % NB
                    if ji >= NB:
                        sd[ji - NB].wait()
                    gd[ji] = pltpu.async_copy(y_hbm.at[srcv.at[ib, ji]],
                                              stage.at[b], gsem.at[b])
                if j >= 0:
                    gd[j].wait()
                    sd[j] = pltpu.async_copy(stage.at[j % NB],
                                             acc.at[locv.at[ib, j]],
                                             ssem.at[j % NB], add=True)
            for j in range(G - NB, G):
                sd[j].wait()

        plsc.subcore_barrier()

        la = [None] * NUP
        ld = [None] * NUP
        wr = [None] * NUP
        off0 = s * ROWS_PT
        la[0] = pltpu.async_copy(acc.at[pl.ds(off0, ZR)],
                                 stage.at[0, pl.ds(0, ZR)], gsem.at[0])
        ld[0] = pltpu.async_copy(d2_hbm.at[pl.ds(off0, ZR)],
                                 stage.at[1, pl.ds(0, ZR)], gsem.at[1])
        for t in range(NUP):
            h = (t % 2) * HOFF
            la[t].wait()
            ld[t].wait()
            if t + 1 < NUP:
                off2 = off0 + (t + 1) * ZR
                h2 = ((t + 1) % 2) * HOFF
                la[t + 1] = pltpu.async_copy(acc.at[pl.ds(off2, ZR)],
                                             stage.at[0, pl.ds(h2, ZR)],
                                             gsem.at[0])
                ld[t + 1] = pltpu.async_copy(d2_hbm.at[pl.ds(off2, ZR)],
                                             stage.at[1, pl.ds(h2, ZR)],
                                             gsem.at[1])
            if t >= 2:
                wr[t - 2].wait()

            @pl.loop(0, ZR)
            def _(r):
                for q in range(HID2 // 16):
                    av = stage[0, h + r, pl.ds(q * 16, 16)]
                    dv = stage[1, h + r, pl.ds(q * 16, 16)]
                    stage[2, h + r, pl.ds(q * 16, 16)] = av * dv

            wr[t] = pltpu.async_copy(
                stage.at[2, pl.ds(h, ZR)],
                out_hbm.at[pl.ds(roff + off0 + t * ZR, ZR)],
                ssem.at[t % 2])
        wr[NUP - 2].wait()
        wr[NUP - 1].wait()

    return k(y2, src2d, loc2d, d2)


def _ln_block(x, g, b, eps=1e-5):
    m = jnp.mean(x, axis=-1, keepdims=True)
    v = jnp.mean((x - m) * (x - m), axis=-1, keepdims=True)
    return (x - m) / jnp.sqrt(v + eps) * g + b


def _tc_item(feat, emb_i, degA, degB, W1, b1, g1, be1, W2, b2, g2, be2,
             W3, b3, mw):
    B = 1000
    grid = N_ITEMS // B

    def body(feat_ref, emb_ref, dA_ref, dB_ref, W1r, b1r, g1r, be1r, W2r,
             b2r, g2r, be2r, W3r, b3r, mwr, out0_ref, ylo_ref, yhi_ref,
             d2_ref, deg_ref):
        h = jnp.dot(feat_ref[...], W1r[...],
                    preferred_element_type=jnp.float32) + b1r[...]
        h = jnp.maximum(_ln_block(h, g1r[...], be1r[...]), 0.0)
        h = jnp.dot(h, W2r[...], preferred_element_type=jnp.float32) + b2r[...]
        h = jnp.maximum(_ln_block(h, g2r[...], be2r[...]), 0.0)
        h = jnp.dot(h, W3r[...], preferred_element_type=jnp.float32) + b3r[...]
        nrm = jnp.sqrt(jnp.sum(h * h, axis=-1, keepdims=True))
        meta = h / jnp.clip(nrm, 1e-12, None)
        e0 = emb_ref[...] + mwr[0, 0] * meta
        deg = dA_ref[...] + dB_ref[...]
        dis = jnp.where(deg > 0, lax.rsqrt(deg), 0.0)
        out0_ref[...] = e0 * ALPHA
        y0 = e0 * dis
        ylo_ref[...] = y0[:, :HID2]
        yhi_ref[...] = y0[:, HID2:]
        d2_ref[...] = jnp.broadcast_to(dis * dis, (B, HID2))
        deg_ref[...] = deg

    full = lambda shp: pl.BlockSpec(shp, lambda i: (0, 0))
    return pl.pallas_call(
        body,
        grid=(grid,),
        in_specs=[
            pl.BlockSpec((B, FEAT), lambda i: (i, 0)),
            pl.BlockSpec((B, HID), lambda i: (i, 0)),
            pl.BlockSpec((B, 1), lambda i: (i, 0)),
            pl.BlockSpec((B, 1), lambda i: (i, 0)),
            full((FEAT, 512)), full((1, 512)), full((1, 512)), full((1, 512)),
            full((512, HID)), full((1, HID)), full((1, HID)), full((1, HID)),
            full((HID, HID)), full((1, HID)), full((1, 1)),
        ],
        out_specs=[pl.BlockSpec((B, HID), lambda i: (i, 0)),
                   pl.BlockSpec((B, HID2), lambda i: (i, 0)),
                   pl.BlockSpec((B, HID2), lambda i: (i, 0)),
                   pl.BlockSpec((B, HID2), lambda i: (i, 0)),
                   pl.BlockSpec((B, 1), lambda i: (i, 0))],
        out_shape=[jax.ShapeDtypeStruct((N_ITEMS, HID), jnp.float32),
                   jax.ShapeDtypeStruct((N_ITEMS, HID2), jnp.float32),
                   jax.ShapeDtypeStruct((N_ITEMS, HID2), jnp.float32),
                   jax.ShapeDtypeStruct((N_ITEMS, HID2), jnp.float32),
                   jax.ShapeDtypeStruct((N_ITEMS, 1), jnp.float32)],
    )(feat, emb_i, degA, degB, W1, b1.reshape(1, -1), g1.reshape(1, -1),
      be1.reshape(1, -1), W2, b2.reshape(1, -1), g2.reshape(1, -1),
      be2.reshape(1, -1), W3, b3.reshape(1, -1), mw.reshape(1, 1))


def _tc_user(emb_u, degA, degB):
    B = 1000
    grid = N_USERS // B

    def body(emb_ref, dA_ref, dB_ref, out0_ref, ylo_ref, yhi_ref, d2_ref,
             deg_ref):
        e0 = emb_ref[...]
        deg = dA_ref[...] + dB_ref[...]
        dis = jnp.where(deg > 0, lax.rsqrt(deg), 0.0)
        out0_ref[...] = e0 * ALPHA
        y0 = e0 * dis
        ylo_ref[...] = y0[:, :HID2]
        yhi_ref[...] = y0[:, HID2:]
        d2_ref[...] = jnp.broadcast_to(dis * dis, (B, HID2))
        deg_ref[...] = deg

    return pl.pallas_call(
        body,
        grid=(grid,),
        in_specs=[pl.BlockSpec((B, HID), lambda i: (i, 0)),
                  pl.BlockSpec((B, 1), lambda i: (i, 0)),
                  pl.BlockSpec((B, 1), lambda i: (i, 0))],
        out_specs=[pl.BlockSpec((B, HID), lambda i: (i, 0)),
                   pl.BlockSpec((B, HID2), lambda i: (i, 0)),
                   pl.BlockSpec((B, HID2), lambda i: (i, 0)),
                   pl.BlockSpec((B, HID2), lambda i: (i, 0)),
                   pl.BlockSpec((B, 1), lambda i: (i, 0))],
        out_shape=[jax.ShapeDtypeStruct((N_USERS, HID), jnp.float32),
                   jax.ShapeDtypeStruct((N_USERS, HID2), jnp.float32),
                   jax.ShapeDtypeStruct((N_USERS, HID2), jnp.float32),
                   jax.ShapeDtypeStruct((N_USERS, HID2), jnp.float32),
                   jax.ShapeDtypeStruct((N_USERS, 1), jnp.float32)],
    )(emb_u, degA, degB)


def _tc_fin(out0, deg, ylos, yhis):
    B = 1000
    grid = N_NODES // B

    def body(out0_ref, deg_ref, l1, l2, l3, h1, h2, h3, out_ref):
        sq = jnp.sqrt(deg_ref[...])
        lo = (l1[...] + l2[...] + l3[...]) * sq
        hi = (h1[...] + h2[...] + h3[...]) * sq
        out_ref[...] = out0_ref[...] + ALPHA * jnp.concatenate([lo, hi],
                                                               axis=1)

    bs64 = pl.BlockSpec((B, HID), lambda i: (i, 0))
    bs32 = pl.BlockSpec((B, HID2), lambda i: (i, 0))
    bs1 = pl.BlockSpec((B, 1), lambda i: (i, 0))
    return pl.pallas_call(
        body,
        grid=(grid,),
        in_specs=[bs64, bs1, bs32, bs32, bs32, bs32, bs32, bs32],
        out_specs=bs64,
        out_shape=jax.ShapeDtypeStruct((N_NODES, HID), jnp.float32),
    )(out0, deg, *ylos, *yhis)


def kernel(edge_index, item_features, emb, W1, b1, g1, be1, W2, b2, g2, be2,
           W3, b3, meta_weight):
    src = edge_index[0].astype(jnp.int32)
    dst = edge_index[1].astype(jnp.int32)
    pad = E_PAD - N_EDGES
    src_p = jnp.concatenate([src, jnp.zeros((pad,), jnp.int32)])
    dst_p = jnp.concatenate([dst, jnp.full((pad,), -1, jnp.int32)])

    degp, loc2d = _sc_deg(dst_p.reshape(EROWS_D, KD))
    degA = degp[:N_NODES].reshape(N_NODES, 1)
    degB = degp[ACCF:ACCF + N_NODES].reshape(N_NODES, 1)

    out0_i, ylo_i, yhi_i, d2_i, deg_i = _tc_item(
        item_features, emb[N_USERS:], degA[N_USERS:], degB[N_USERS:],
        W1, b1, g1, be1, W2, b2, g2, be2, W3, b3, meta_weight)
    out0_u, ylo_u, yhi_u, d2_u, deg_u = _tc_user(
        emb[:N_USERS], degA[:N_USERS], degB[:N_USERS])

    out0 = jnp.concatenate([out0_u, out0_i])
    deg = jnp.concatenate([deg_u, deg_i])
    padrows = ((0, ACCF - N_NODES), (0, 0))
    y2 = jnp.concatenate([
        jnp.pad(jnp.concatenate([ylo_u, ylo_i]), padrows),
        jnp.pad(jnp.concatenate([yhi_u, yhi_i]), padrows)])
    d2 = jnp.pad(jnp.concatenate([d2_u, d2_i]), padrows)

    src2d = src_p.reshape(EROWS, K)
    loc2d_p = loc2d.reshape(EROWS, K)
    ylos, yhis = [], []
    for _ in range(N_LAYERS):
        y2 = _sc_prop(y2, src2d, loc2d_p, d2)
        ylos.append(y2[:N_NODES])
        yhis.append(y2[ACCF:ACCF + N_NODES])
    return _tc_fin(out0, deg, ylos, yhis)


# speedup vs baseline: 2.5713x; 1.0034x over previous
import functools

import jax
import jax.numpy as jnp
from jax import lax
from jax.experimental import pallas as pl
from jax.experimental.pallas import tpu as pltpu
from jax.experimental.pallas import tpu_sc as plsc

N_NODES = 50000
N_USERS = 25000
N_ITEMS = 25000
FEAT = 128
HID = 64
HID2 = HID // 2
N_LAYERS = 3
N_EDGES = 800000
ALPHA = 1.0 / (N_LAYERS + 1)

NC = 2
NS = 16
E_PAD = 835584

ROWS_PT = 3136
ACCF = NS * ROWS_PT

KD = 512
EROWS_D = E_PAD // KD
RPT_D = EROWS_D // (NC * NS)
GD = 17
NSUP_D = 3

K = 128
CPT = 408
G = 24
NSUP = 17
EROWS = E_PAD // K
NB = 4
LA = 3
ZR = 56
HOFF = 64
NUP = ROWS_PT // ZR


def _sc_deg(dst2d):
    mesh = plsc.VectorSubcoreMesh(core_axis_name="c", subcore_axis_name="s",
                                  num_cores=NC, num_subcores=NS)

    @functools.partial(
        pl.kernel,
        out_type=(jax.ShapeDtypeStruct((NC * ACCF,), jnp.float32),
                  jax.ShapeDtypeStruct((EROWS_D, KD), jnp.int32)),
        mesh=mesh,
        scratch_types=[
            pltpu.VMEM((GD, KD), jnp.int32),
            pltpu.VMEM((KD,), jnp.float32),
            pltpu.VMEM((ROWS_PT,), jnp.float32),
            pltpu.VMEM_SHARED((ACCF,), jnp.float32),
            pltpu.SemaphoreType.DMA((NB,)),
        ],
        compiler_params=pltpu.CompilerParams(use_tc_tiling_on_sc=False),
    )
    def k(dst_hbm, out_hbm, loc_hbm, locv, ones, zb, acc, ssem):
        c = lax.axis_index("c")
        s = lax.axis_index("s")
        w = c * NS + s

        @pl.loop(0, KD // 16)
        def _(i):
            ones[pl.ds(i * 16, 16)] = jnp.full((16,), 1.0, jnp.float32)

        @pl.loop(0, ROWS_PT // 16)
        def _(i):
            zb[pl.ds(i * 16, 16)] = jnp.zeros((16,), jnp.float32)

        pltpu.sync_copy(zb, acc.at[pl.ds(s * ROWS_PT, ROWS_PT)])
        plsc.subcore_barrier()

        @pl.loop(0, NSUP_D)
        def _(g):
            base = w * RPT_D + g * GD
            pltpu.sync_copy(dst_hbm.at[pl.ds(base, GD)], locv)

            @pl.loop(0, GD)
            def _(r):
                for q in range(KD // 16):
                    d = locv[r, pl.ds(q * 16, 16)]
                    locv[r, pl.ds(q * 16, 16)] = jnp.where(
                        d >= 0, d, N_NODES)

            pltpu.sync_copy(locv, loc_hbm.at[pl.ds(base, GD)])
            sd = [None] * GD
            for j in range(GD):
                if j >= NB:
                    sd[j - NB].wait()
                sd[j] = pltpu.async_copy(ones, acc.at[locv.at[j]],
                                         ssem.at[j % NB], add=True)
            for j in range(GD - NB, GD):
                sd[j].wait()

        plsc.subcore_barrier()
        pltpu.sync_copy(acc.at[pl.ds(s * ROWS_PT, ROWS_PT)], zb)
        pltpu.sync_copy(zb, out_hbm.at[pl.ds(c * ACCF + s * ROWS_PT,
                                             ROWS_PT)])

    return k(dst2d)


def _sc_prop(y2, src2d, loc2d, d2):
    mesh = plsc.VectorSubcoreMesh(core_axis_name="c", subcore_axis_name="s",
                                  num_cores=NC, num_subcores=NS)

    @functools.partial(
        pl.kernel,
        out_type=jax.ShapeDtypeStruct((NC * ACCF, HID2), jnp.float32),
        mesh=mesh,
        scratch_types=[
            pltpu.VMEM((2, G, K), jnp.int32),
            pltpu.VMEM((2, G, K), jnp.int32),
            pltpu.VMEM((NB, K, HID2), jnp.float32),
            pltpu.VMEM_SHARED((ACCF, HID2), jnp.float32),
            pltpu.SemaphoreType.DMA((NB,)),
            pltpu.SemaphoreType.DMA((NB,)),
            pltpu.SemaphoreType.DMA,
            pltpu.SemaphoreType.DMA,
        ],
        compiler_params=pltpu.CompilerParams(use_tc_tiling_on_sc=False),
    )
    def k(y_hbm, src_hbm, loc_hbm, d2_hbm, out_hbm, srcv, locv, stage, acc,
          gsem, ssem, isemA, isemB):
        c = lax.axis_index("c")
        s = lax.axis_index("s")
        roff = c * ACCF

        @pl.loop(0, ZR)
        def _(r):
            for q in range(HID2 // 16):
                stage[2, r, pl.ds(q * 16, 16)] = jnp.zeros((16,),
                                                           jnp.float32)

        for t in range(NUP):
            pltpu.sync_copy(stage.at[2, pl.ds(0, ZR)],
                            acc.at[pl.ds(s * ROWS_PT + t * ZR, ZR)])
        plsc.subcore_barrier()

        pltpu.sync_copy(src_hbm.at[pl.ds(s * CPT, G)], srcv.at[0])
        pltpu.sync_copy(loc_hbm.at[pl.ds(s * CPT, G)], locv.at[0])

        @pl.loop(0, G)
        def _(r):
            for q in range(K // 16):
                sv = srcv[0, r, pl.ds(q * 16, 16)]
                srcv[0, r, pl.ds(q * 16, 16)] = sv + roff

        for b in range(LA):
            pltpu.async_copy(y_hbm.at[srcv.at[0, b]], stage.at[b],
                             gsem.at[b])

        @pl.loop(0, NSUP)
        def _(g):
            ib = lax.rem(g, 2)
            @pl.when(g < NSUP - 1)
            def _():
                nbase = s * CPT + (g + 1) * G
                pltpu.async_copy(src_hbm.at[pl.ds(nbase, G)],
                                 srcv.at[1 - ib], isemA)
                pltpu.async_copy(loc_hbm.at[pl.ds(nbase, G)],
                                 locv.at[1 - ib], isemB)

            gd = [None] * G
            sd = [None] * G
            for j in range(G):
                ji = j + LA
                if ji < G:
                    b = ji % NB
                    if ji >= NB:
                        sd[ji - NB].wait()
                    gd[ji] = pltpu.async_copy(y_hbm.at[srcv.at[ib, ji]],
                                              stage.at[b], gsem.at[b])
                if j < LA:
                    pltpu.make_async_copy(y_hbm.at[pl.ds(0, K)],
                                          stage.at[j % NB],
                                          gsem.at[j % NB]).wait()
                else:
                    gd[j].wait()
                sd[j] = pltpu.async_copy(stage.at[j % NB],
                                         acc.at[locv.at[ib, j]],
                                         ssem.at[j % NB], add=True)
            for j in range(G - NB, G):
                sd[j].wait()

            @pl.when(g < NSUP - 1)
            def _():
                pltpu.make_async_copy(src_hbm.at[pl.ds(0, G)],
                                      srcv.at[1 - ib], isemA).wait()
                pltpu.make_async_copy(loc_hbm.at[pl.ds(0, G)],
                                      locv.at[1 - ib], isemB).wait()

                @pl.loop(0, G)
                def _(r):
                    for q in range(K // 16):
                        sv = srcv[1 - ib, r, pl.ds(q * 16, 16)]
                        srcv[1 - ib, r, pl.ds(q * 16, 16)] = sv + roff

                for b in range(LA):
                    pltpu.async_copy(y_hbm.at[srcv.at[1 - ib, b]],
                                     stage.at[b], gsem.at[b])

        plsc.subcore_barrier()

        la = [None] * NUP
        ld = [None] * NUP
        wr = [None] * NUP
        off0 = s * ROWS_PT
        la[0] = pltpu.async_copy(acc.at[pl.ds(off0, ZR)],
                                 stage.at[0, pl.ds(0, ZR)], gsem.at[0])
        ld[0] = pltpu.async_copy(d2_hbm.at[pl.ds(off0, ZR)],
                                 stage.at[1, pl.ds(0, ZR)], gsem.at[1])
        for t in range(NUP):
            h = (t % 2) * HOFF
            la[t].wait()
            ld[t].wait()
            if t + 1 < NUP:
                off2 = off0 + (t + 1) * ZR
                h2 = ((t + 1) % 2) * HOFF
                la[t + 1] = pltpu.async_copy(acc.at[pl.ds(off2, ZR)],
                                             stage.at[0, pl.ds(h2, ZR)],
                                             gsem.at[0])
                ld[t + 1] = pltpu.async_copy(d2_hbm.at[pl.ds(off2, ZR)],
                                             stage.at[1, pl.ds(h2, ZR)],
                                             gsem.at[1])
            if t >= 2:
                wr[t - 2].wait()

            @pl.loop(0, ZR)
            def _(r):
                for q in range(HID2 // 16):
                    av = stage[0, h + r, pl.ds(q * 16, 16)]
                    dv = stage[1, h + r, pl.ds(q * 16, 16)]
                    stage[2, h + r, pl.ds(q * 16, 16)] = av * dv

            wr[t] = pltpu.async_copy(
                stage.at[2, pl.ds(h, ZR)],
                out_hbm.at[pl.ds(roff + off0 + t * ZR, ZR)],
                ssem.at[t % 2])
        wr[NUP - 2].wait()
        wr[NUP - 1].wait()

    return k(y2, src2d, loc2d, d2)


def _ln_block(x, g, b, eps=1e-5):
    m = jnp.mean(x, axis=-1, keepdims=True)
    v = jnp.mean((x - m) * (x - m), axis=-1, keepdims=True)
    return (x - m) / jnp.sqrt(v + eps) * g + b


def _tc_item(feat, emb_i, degA, degB, W1, b1, g1, be1, W2, b2, g2, be2,
             W3, b3, mw):
    B = 1000
    grid = N_ITEMS // B

    def body(feat_ref, emb_ref, dA_ref, dB_ref, W1r, b1r, g1r, be1r, W2r,
             b2r, g2r, be2r, W3r, b3r, mwr, out0_ref, ylo_ref, yhi_ref,
             d2_ref, deg_ref):
        h = jnp.dot(feat_ref[...], W1r[...],
                    preferred_element_type=jnp.float32) + b1r[...]
        h = jnp.maximum(_ln_block(h, g1r[...], be1r[...]), 0.0)
        h = jnp.dot(h, W2r[...], preferred_element_type=jnp.float32) + b2r[...]
        h = jnp.maximum(_ln_block(h, g2r[...], be2r[...]), 0.0)
        h = jnp.dot(h, W3r[...], preferred_element_type=jnp.float32) + b3r[...]
        nrm = jnp.sqrt(jnp.sum(h * h, axis=-1, keepdims=True))
        meta = h / jnp.clip(nrm, 1e-12, None)
        e0 = emb_ref[...] + mwr[0, 0] * meta
        deg = dA_ref[...] + dB_ref[...]
        dis = jnp.where(deg > 0, lax.rsqrt(deg), 0.0)
        out0_ref[...] = e0 * ALPHA
        y0 = e0 * dis
        ylo_ref[...] = y0[:, :HID2]
        yhi_ref[...] = y0[:, HID2:]
        d2_ref[...] = jnp.broadcast_to(dis * dis, (B, HID2))
        deg_ref[...] = deg

    full = lambda shp: pl.BlockSpec(shp, lambda i: (0, 0))
    return pl.pallas_call(
        body,
        grid=(grid,),
        in_specs=[
            pl.BlockSpec((B, FEAT), lambda i: (i, 0)),
            pl.BlockSpec((B, HID), lambda i: (i, 0)),
            pl.BlockSpec((B, 1), lambda i: (i, 0)),
            pl.BlockSpec((B, 1), lambda i: (i, 0)),
            full((FEAT, 512)), full((1, 512)), full((1, 512)), full((1, 512)),
            full((512, HID)), full((1, HID)), full((1, HID)), full((1, HID)),
            full((HID, HID)), full((1, HID)), full((1, 1)),
        ],
        out_specs=[pl.BlockSpec((B, HID), lambda i: (i, 0)),
                   pl.BlockSpec((B, HID2), lambda i: (i, 0)),
                   pl.BlockSpec((B, HID2), lambda i: (i, 0)),
                   pl.BlockSpec((B, HID2), lambda i: (i, 0)),
                   pl.BlockSpec((B, 1), lambda i: (i, 0))],
        out_shape=[jax.ShapeDtypeStruct((N_ITEMS, HID), jnp.float32),
                   jax.ShapeDtypeStruct((N_ITEMS, HID2), jnp.float32),
                   jax.ShapeDtypeStruct((N_ITEMS, HID2), jnp.float32),
                   jax.ShapeDtypeStruct((N_ITEMS, HID2), jnp.float32),
                   jax.ShapeDtypeStruct((N_ITEMS, 1), jnp.float32)],
    )(feat, emb_i, degA, degB, W1, b1.reshape(1, -1), g1.reshape(1, -1),
      be1.reshape(1, -1), W2, b2.reshape(1, -1), g2.reshape(1, -1),
      be2.reshape(1, -1), W3, b3.reshape(1, -1), mw.reshape(1, 1))


def _tc_user(emb_u, degA, degB):
    B = 1000
    grid = N_USERS // B

    def body(emb_ref, dA_ref, dB_ref, out0_ref, ylo_ref, yhi_ref, d2_ref,
             deg_ref):
        e0 = emb_ref[...]
        deg = dA_ref[...] + dB_ref[...]
        dis = jnp.where(deg > 0, lax.rsqrt(deg), 0.0)
        out0_ref[...] = e0 * ALPHA
        y0 = e0 * dis
        ylo_ref[...] = y0[:, :HID2]
        yhi_ref[...] = y0[:, HID2:]
        d2_ref[...] = jnp.broadcast_to(dis * dis, (B, HID2))
        deg_ref[...] = deg

    return pl.pallas_call(
        body,
        grid=(grid,),
        in_specs=[pl.BlockSpec((B, HID), lambda i: (i, 0)),
                  pl.BlockSpec((B, 1), lambda i: (i, 0)),
                  pl.BlockSpec((B, 1), lambda i: (i, 0))],
        out_specs=[pl.BlockSpec((B, HID), lambda i: (i, 0)),
                   pl.BlockSpec((B, HID2), lambda i: (i, 0)),
                   pl.BlockSpec((B, HID2), lambda i: (i, 0)),
                   pl.BlockSpec((B, HID2), lambda i: (i, 0)),
                   pl.BlockSpec((B, 1), lambda i: (i, 0))],
        out_shape=[jax.ShapeDtypeStruct((N_USERS, HID), jnp.float32),
                   jax.ShapeDtypeStruct((N_USERS, HID2), jnp.float32),
                   jax.ShapeDtypeStruct((N_USERS, HID2), jnp.float32),
                   jax.ShapeDtypeStruct((N_USERS, HID2), jnp.float32),
                   jax.ShapeDtypeStruct((N_USERS, 1), jnp.float32)],
    )(emb_u, degA, degB)


def _tc_fin(out0, deg, ylos, yhis):
    B = 1000
    grid = N_NODES // B

    def body(out0_ref, deg_ref, l1, l2, l3, h1, h2, h3, out_ref):
        sq = jnp.sqrt(deg_ref[...])
        lo = (l1[...] + l2[...] + l3[...]) * sq
        hi = (h1[...] + h2[...] + h3[...]) * sq
        out_ref[...] = out0_ref[...] + ALPHA * jnp.concatenate([lo, hi],
                                                               axis=1)

    bs64 = pl.BlockSpec((B, HID), lambda i: (i, 0))
    bs32 = pl.BlockSpec((B, HID2), lambda i: (i, 0))
    bs1 = pl.BlockSpec((B, 1), lambda i: (i, 0))
    return pl.pallas_call(
        body,
        grid=(grid,),
        in_specs=[bs64, bs1, bs32, bs32, bs32, bs32, bs32, bs32],
        out_specs=bs64,
        out_shape=jax.ShapeDtypeStruct((N_NODES, HID), jnp.float32),
    )(out0, deg, *ylos, *yhis)


def kernel(edge_index, item_features, emb, W1, b1, g1, be1, W2, b2, g2, be2,
           W3, b3, meta_weight):
    src = edge_index[0].astype(jnp.int32)
    dst = edge_index[1].astype(jnp.int32)
    pad = E_PAD - N_EDGES
    src_p = jnp.concatenate([src, jnp.zeros((pad,), jnp.int32)])
    dst_p = jnp.concatenate([dst, jnp.full((pad,), -1, jnp.int32)])

    degp, loc2d = _sc_deg(dst_p.reshape(EROWS_D, KD))
    degA = degp[:N_NODES].reshape(N_NODES, 1)
    degB = degp[ACCF:ACCF + N_NODES].reshape(N_NODES, 1)

    out0_i, ylo_i, yhi_i, d2_i, deg_i = _tc_item(
        item_features, emb[N_USERS:], degA[N_USERS:], degB[N_USERS:],
        W1, b1, g1, be1, W2, b2, g2, be2, W3, b3, meta_weight)
    out0_u, ylo_u, yhi_u, d2_u, deg_u = _tc_user(
        emb[:N_USERS], degA[:N_USERS], degB[:N_USERS])

    out0 = jnp.concatenate([out0_u, out0_i])
    deg = jnp.concatenate([deg_u, deg_i])
    padrows = ((0, ACCF - N_NODES), (0, 0))
    y2 = jnp.concatenate([
        jnp.pad(jnp.concatenate([ylo_u, ylo_i]), padrows),
        jnp.pad(jnp.concatenate([yhi_u, yhi_i]), padrows)])
    d2 = jnp.pad(jnp.concatenate([d2_u, d2_i]), padrows)

    src2d = src_p.reshape(EROWS, K)
    loc2d_p = loc2d.reshape(EROWS, K)
    ylos, yhis = [], []
    for _ in range(N_LAYERS):
        y2 = _sc_prop(y2, src2d, loc2d_p, d2)
        ylos.append(y2[:N_NODES])
        yhis.append(y2[ACCF:ACCF + N_NODES])
    return _tc_fin(out0, deg, ylos, yhis)
